# baseline (speedup 1.0000x reference)
import functools

import jax
import jax.numpy as jnp
from jax import lax
from jax.experimental import pallas as pl
from jax.experimental.pallas import tpu as pltpu
from jax.experimental.pallas import tpu_sc as plsc

BATCH = 16384
D = 128
NUM_EMB = 201
INDEX_OFFSET = 100
NC = 2
NS = 16
NW = NC * NS
CHUNK = 128
ROWS_PER_W = BATCH // (NW * CHUNK)
L = 16

_BCAST_DNUMS = lax.GatherDimensionNumbers(
    offset_dims=(), collapsed_slice_dims=(0,), start_index_map=(0,)
)


def _lane_broadcast(vec, r):
    return lax.gather(
        vec,
        jnp.full((L, 1), r, jnp.int32),
        dimension_numbers=_BCAST_DNUMS,
        slice_sizes=(1,),
        mode=lax.GatherScatterMode.PROMISE_IN_BOUNDS,
    )


def kernel(x, table):
    x2 = x.reshape(NW, ROWS_PER_W * CHUNK)
    mesh = plsc.VectorSubcoreMesh(core_axis_name="c", subcore_axis_name="s")

    @functools.partial(
        pl.kernel,
        mesh=mesh,
        out_type=jax.ShapeDtypeStruct((NW, ROWS_PER_W, CHUNK * D), jnp.float32),
        scratch_types=[
            pltpu.VMEM((ROWS_PER_W * CHUNK,), jnp.int32),
            pltpu.VMEM((NUM_EMB, D), jnp.float32),
        ]
        + [pltpu.VMEM((CHUNK * D,), jnp.float32)] * ROWS_PER_W
        + [pltpu.SemaphoreType.DMA],
        compiler_params=pltpu.CompilerParams(needs_layout_passes=False),
    )
    def emb(x_hbm, tab_hbm, out_hbm, idx_v, tab_v, *rest):
        rows_bufs, wsem = rest[:ROWS_PER_W], rest[ROWS_PER_W]
        wid = lax.axis_index("s") * NC + lax.axis_index("c")
        pltpu.sync_copy(x_hbm.at[wid], idx_v)
        pltpu.sync_copy(tab_hbm, tab_v)
        io = lax.iota(jnp.int32, L)
        coladdr = [io + c8 * L for c8 in range(D // L)]

        writes = []
        for i in range(ROWS_PER_W):
            rows_i = rows_bufs[i]

            def block(b, carry, i=i, rows_i=rows_i):
                off = i * CHUNK + b * L
                tvec = idx_v[pl.ds(off, L)] + INDEX_OFFSET
                dst0 = b * (L * D)
                for r in range(L):
                    t = _lane_broadcast(tvec, r)
                    for c8 in range(D // L):
                        vals = plsc.load_gather(tab_v, [t, coladdr[c8]])
                        rows_i[pl.ds(dst0 + r * D + c8 * L, L)] = vals
                return carry

            lax.fori_loop(0, CHUNK // L, block, 0)
            writes.append(pltpu.async_copy(rows_i, out_hbm.at[wid, i], wsem))
        for w in writes:
            w.wait()

    return emb(x2, table).reshape(BATCH, D)


# speedup vs baseline: 2.1393x; 2.1393x over previous
import functools

import jax
import jax.numpy as jnp
from jax import lax
from jax.experimental import pallas as pl
from jax.experimental.pallas import tpu as pltpu
from jax.experimental.pallas import tpu_sc as plsc

BATCH = 16384
D = 128
INDEX_OFFSET = 100
NC = 2
NS = 16
NW = NC * NS
CHUNK = 128
ROWS_PER_W = BATCH // (NW * CHUNK)


def kernel(x, table):
    x3 = x.reshape(NW, ROWS_PER_W, CHUNK)
    mesh = plsc.VectorSubcoreMesh(core_axis_name="c", subcore_axis_name="s")

    @functools.partial(
        pl.kernel,
        mesh=mesh,
        out_type=jax.ShapeDtypeStruct((NW, ROWS_PER_W, CHUNK, D), jnp.float32),
        scratch_types=[
            pltpu.VMEM((ROWS_PER_W, CHUNK), jnp.int32),
            pltpu.VMEM((ROWS_PER_W, CHUNK, D), jnp.float32),
            pltpu.VMEM_SHARED((201, D), jnp.float32),
        ]
        + [pltpu.SemaphoreType.DMA] * ROWS_PER_W
        + [pltpu.SemaphoreType.DMA],
    )
    def emb(x_hbm, table_hbm, out_hbm, idx_v, rows_v, tab_sp, *sems):
        gsems, wsem = sems[:ROWS_PER_W], sems[ROWS_PER_W]
        sid = lax.axis_index("s")
        wid = sid * NC + lax.axis_index("c")

        @pl.when(sid == 0)
        def _():
            pltpu.sync_copy(table_hbm, tab_sp)

        pltpu.sync_copy(x_hbm.at[wid], idx_v)
        for i in range(ROWS_PER_W):
            row = idx_v.at[i]
            for j in range(CHUNK // 16):
                s = pl.ds(j * 16, 16)
                row[s] = row[s] + INDEX_OFFSET
        plsc.subcore_barrier()
        gathers = [
            pltpu.async_copy(tab_sp.at[idx_v.at[i]], rows_v.at[i], gsems[i])
            for i in range(ROWS_PER_W)
        ]
        writes = []
        for i in range(ROWS_PER_W):
            gathers[i].wait()
            writes.append(
                pltpu.async_copy(rows_v.at[i], out_hbm.at[wid, i], wsem)
            )
        for w in writes:
            w.wait()

    return emb(x3, table).reshape(BATCH, D)
